# baseline (speedup 1.0000x reference)
import functools

import numpy as np
import jax
import jax.numpy as jnp
from jax import lax
from jax.experimental import pallas as pl
from jax.experimental.pallas import tpu as pltpu
from jax.experimental.pallas import tpu_sc as plsc

_N = 16384
_NNZ = 262144
_DPROB = 0.5
_SCALE = np.float32(1.0 / (1.0001 - _DPROB))


def _tf2x32_np(k0, k1, x0, x1):
    rot1 = (13, 15, 26, 6)
    rot2 = (17, 29, 16, 24)
    k0 = np.uint32(k0)
    k1 = np.uint32(k1)
    ks = (k0, k1, np.uint32(k0 ^ k1 ^ np.uint32(0x1BD11BDA)))
    x0 = x0.astype(np.uint32) + k0
    x1 = x1.astype(np.uint32) + k1

    def rnds(x0, x1, rots):
        for r in rots:
            x0 = x0 + x1
            x1 = (x1 << np.uint32(r)) | (x1 >> np.uint32(32 - r))
            x1 = x0 ^ x1
        return x0, x1

    for i in range(5):
        x0, x1 = rnds(x0, x1, rot1 if i % 2 == 0 else rot2)
        x0 = x0 + ks[(i + 1) % 3]
        x1 = x1 + ks[(i + 2) % 3] + np.uint32(i + 1)
    return x0, x1


def _fixed_mask_np(nnz, dprob):
    o0, o1 = _tf2x32_np(0, 0, np.array([0], np.uint32), np.array([123], np.uint32))
    b0, b1 = _tf2x32_np(o0[0], o1[0], np.zeros(nnz, np.uint32),
                        np.arange(nnz, dtype=np.uint32))
    bits = b0 ^ b1
    float_bits = (bits >> np.uint32(9)) | np.uint32(0x3F800000)
    u = np.maximum(np.float32(0.0), float_bits.view(np.float32) - np.float32(1.0))
    return np.floor(u + np.float32(1.0 - dprob)).astype(bool)


_msk = _fixed_mask_np(_NNZ, _DPROB)
_sel = np.nonzero(_msk)[0].astype(np.int64)
_K = int(_sel.shape[0])

_NW = 32
_OPT = 4224
_PAD = _NW * _OPT
_STRIDE = 8448
_OFF = -1024
_IN_LEN = 9344
_HL = 4608
_J1 = 112

_TAIL = _K - (_NW - 1) * _OPT
_PADC = ((_K + 127) // 128) * 128
_CTAIL = _PADC - (_NW - 1) * _OPT
assert 0 < _TAIL <= _OPT and (_NW - 1) * _OPT % 128 == 0 and _CTAIL % 128 == 0

_selp = np.concatenate([_sel, np.full(_PAD - _K, _sel[-1], np.int64)])
_in_lo_np = np.clip(_OFF + np.arange(_NW) * _STRIDE, 0, _NNZ - _IN_LEN)
_lidx_np = (_selp.reshape(_NW, _OPT) - _in_lo_np[:, None]).astype(np.int32)
assert _lidx_np.min() >= 0 and _lidx_np.max() < _IN_LEN
_LIDX = _lidx_np.reshape(_PAD)


@functools.partial(
    pl.kernel,
    mesh=plsc.VectorSubcoreMesh(core_axis_name="c", subcore_axis_name="s"),
    compiler_params=pltpu.CompilerParams(needs_layout_passes=False),
    out_type=[
        jax.ShapeDtypeStruct((2, _PADC), jnp.int32),
        jax.ShapeDtypeStruct((_K,), jnp.float32),
    ],
    scratch_types=[
        pltpu.VMEM((2, _IN_LEN), jnp.int32),
        pltpu.VMEM((_IN_LEN,), jnp.float32),
        pltpu.VMEM((_OPT,), jnp.int32),
        pltpu.VMEM((2, _OPT), jnp.int32),
        pltpu.VMEM((_OPT,), jnp.float32),
        pltpu.SemaphoreType.DMA,
        pltpu.SemaphoreType.DMA,
        pltpu.SemaphoreType.DMA,
    ],
)
def _compact(xi_hbm, xv_hbm, lidx_hbm, rc_hbm, val_hbm,
             xi_v, xv_v, lidx_v, o01_v, ov_v, sem_in, sem_in2, sem_out):
    c = lax.axis_index("c")
    s = lax.axis_index("s")
    wid = s * 2 + c
    in_lo = pl.multiple_of(jnp.clip(_OFF + wid * _STRIDE, 0, _NNZ - _IN_LEN), 128)
    obase = pl.multiple_of(wid * _OPT, 128)

    in_hi = pl.multiple_of(in_lo + _HL, 128)
    h2 = _IN_LEN - _HL
    cp_xi1 = pltpu.async_copy(xi_hbm.at[:, pl.ds(in_lo, _HL)],
                              xi_v.at[:, pl.ds(0, _HL)], sem_in)
    cp_xv1 = pltpu.async_copy(xv_hbm.at[pl.ds(in_lo, _HL)],
                              xv_v.at[pl.ds(0, _HL)], sem_in)
    cp_li = pltpu.async_copy(lidx_hbm.at[pl.ds(obase, _OPT)], lidx_v, sem_in)
    cp_xi2 = pltpu.async_copy(xi_hbm.at[:, pl.ds(in_hi, h2)],
                              xi_v.at[:, pl.ds(_HL, h2)], sem_in2)
    cp_xv2 = pltpu.async_copy(xv_hbm.at[pl.ds(in_hi, h2)],
                              xv_v.at[pl.ds(_HL, h2)], sem_in2)
    cp_xi1.wait()
    cp_xv1.wait()
    cp_li.wait()

    zeros = jnp.zeros((16,), jnp.int32)
    ones = jnp.ones((16,), jnp.int32)

    def body(i, carry):
        iv = lidx_v[pl.ds(i * 16, 16)]
        ov_v[pl.ds(i * 16, 16)] = plsc.load_gather(xv_v, [iv]) * _SCALE
        o01_v[0, pl.ds(i * 16, 16)] = plsc.load_gather(xi_v, [zeros, iv])
        o01_v[1, pl.ds(i * 16, 16)] = plsc.load_gather(xi_v, [ones, iv])
        return carry

    @pl.when(wid != _NW - 1)
    def _phase1():
        lax.fori_loop(0, _J1, body, 0, unroll=8)

    cp_xi2.wait()
    cp_xv2.wait()

    @pl.when(wid == _NW - 1)
    def _phase1_late():
        lax.fori_loop(0, _J1, body, 0, unroll=8)

    lax.fori_loop(_J1, _OPT // 16, body, 0, unroll=8)

    @pl.when(wid != _NW - 1)
    def _full_write():
        cp_rc = pltpu.async_copy(o01_v, rc_hbm.at[:, pl.ds(obase, _OPT)], sem_out)
        cp_v = pltpu.async_copy(ov_v, val_hbm.at[pl.ds(obase, _OPT)], sem_out)
        cp_rc.wait()
        cp_v.wait()

    @pl.when(wid == _NW - 1)
    def _tail_write():
        cp_rc = pltpu.async_copy(o01_v.at[:, pl.ds(0, _CTAIL)],
                                 rc_hbm.at[:, pl.ds(obase, _CTAIL)], sem_out)
        cp_v = pltpu.async_copy(ov_v.at[pl.ds(0, _TAIL)],
                                val_hbm.at[pl.ds(obase, _TAIL)], sem_out)
        cp_rc.wait()
        cp_v.wait()


def kernel(x_indices, x_values):
    xi = x_indices.astype(jnp.int32)
    lidx = jnp.asarray(_LIDX)
    rc_pad, val = _compact(xi, x_values, lidx)
    return rc_pad[:, :_K].astype(x_indices.dtype), val


# speedup vs baseline: 1.1703x; 1.1703x over previous
import functools

import numpy as np
import jax
import jax.numpy as jnp
from jax import lax
from jax.experimental import pallas as pl
from jax.experimental.pallas import tpu as pltpu
from jax.experimental.pallas import tpu_sc as plsc

_N = 16384
_NNZ = 262144
_DPROB = 0.5
_SCALE = np.float32(1.0 / (1.0001 - _DPROB))


def _tf2x32_np(k0, k1, x0, x1):
    rot1 = (13, 15, 26, 6)
    rot2 = (17, 29, 16, 24)
    k0 = np.uint32(k0)
    k1 = np.uint32(k1)
    ks = (k0, k1, np.uint32(k0 ^ k1 ^ np.uint32(0x1BD11BDA)))
    x0 = x0.astype(np.uint32) + k0
    x1 = x1.astype(np.uint32) + k1

    def rnds(x0, x1, rots):
        for r in rots:
            x0 = x0 + x1
            x1 = (x1 << np.uint32(r)) | (x1 >> np.uint32(32 - r))
            x1 = x0 ^ x1
        return x0, x1

    for i in range(5):
        x0, x1 = rnds(x0, x1, rot1 if i % 2 == 0 else rot2)
        x0 = x0 + ks[(i + 1) % 3]
        x1 = x1 + ks[(i + 2) % 3] + np.uint32(i + 1)
    return x0, x1


def _fixed_mask_np(nnz, dprob):
    o0, o1 = _tf2x32_np(0, 0, np.array([0], np.uint32), np.array([123], np.uint32))
    b0, b1 = _tf2x32_np(o0[0], o1[0], np.zeros(nnz, np.uint32),
                        np.arange(nnz, dtype=np.uint32))
    bits = b0 ^ b1
    float_bits = (bits >> np.uint32(9)) | np.uint32(0x3F800000)
    u = np.maximum(np.float32(0.0), float_bits.view(np.float32) - np.float32(1.0))
    return np.floor(u + np.float32(1.0 - dprob)).astype(bool)


_msk = _fixed_mask_np(_NNZ, _DPROB)
_sel = np.nonzero(_msk)[0].astype(np.int64)
_K = int(_sel.shape[0])

_NW = 32
_OPT = 4224
_PAD = _NW * _OPT
_STRIDE = 8448
_OFF = -1024
_IN_LEN = 9344
_HL = 4608
_J1 = 112

_TAIL = _K - (_NW - 1) * _OPT
_PADC = ((_K + 127) // 128) * 128
_CTAIL = _PADC - (_NW - 1) * _OPT
assert 0 < _TAIL <= _OPT and (_NW - 1) * _OPT % 128 == 0 and _CTAIL % 128 == 0

_selp = np.concatenate([_sel, np.full(_PAD - _K, _sel[-1], np.int64)])
_in_lo_np = np.clip(_OFF + np.arange(_NW) * _STRIDE, 0, _NNZ - _IN_LEN)
_lidx_np = (_selp.reshape(_NW, _OPT) - _in_lo_np[:, None]).astype(np.int32)
assert _lidx_np.min() >= 0 and _lidx_np.max() < _IN_LEN
_LIDX = _lidx_np.reshape(_PAD)


@functools.partial(
    pl.kernel,
    mesh=plsc.VectorSubcoreMesh(core_axis_name="c", subcore_axis_name="s"),
    compiler_params=pltpu.CompilerParams(needs_layout_passes=False),
    out_type=[
        jax.ShapeDtypeStruct((2, _PADC), jnp.int32),
        jax.ShapeDtypeStruct((_K,), jnp.float32),
    ],
    scratch_types=[
        pltpu.VMEM((2, _IN_LEN), jnp.int32),
        pltpu.VMEM((_IN_LEN,), jnp.float32),
        pltpu.VMEM((_OPT,), jnp.int32),
        pltpu.VMEM((2, _OPT), jnp.int32),
        pltpu.VMEM((_OPT,), jnp.float32),
        pltpu.SemaphoreType.DMA,
        pltpu.SemaphoreType.DMA,
        pltpu.SemaphoreType.DMA,
    ],
)
def _compact(xi_hbm, xv_hbm, lidx_hbm, rc_hbm, val_hbm,
             xi_v, xv_v, lidx_v, o01_v, ov_v, sem_in, sem_in2, sem_out):
    c = lax.axis_index("c")
    s = lax.axis_index("s")
    wid = s * 2 + c
    in_lo = pl.multiple_of(jnp.clip(_OFF + wid * _STRIDE, 0, _NNZ - _IN_LEN), 128)
    obase = pl.multiple_of(wid * _OPT, 128)

    cp_xi = pltpu.async_copy(xi_hbm.at[:, pl.ds(in_lo, _IN_LEN)], xi_v, sem_in)
    cp_xv = pltpu.async_copy(xv_hbm.at[pl.ds(in_lo, _IN_LEN)], xv_v, sem_in)
    cp_li = pltpu.async_copy(lidx_hbm.at[pl.ds(obase, _OPT)], lidx_v, sem_in)
    cp_xi.wait()
    cp_xv.wait()
    cp_li.wait()

    zeros = jnp.zeros((16,), jnp.int32)
    ones = jnp.ones((16,), jnp.int32)

    @plsc.parallel_loop(0, _OPT // 16, unroll=8)
    def _gather(i):
        iv = lidx_v[pl.ds(i * 16, 16)]
        ov_v[pl.ds(i * 16, 16)] = plsc.load_gather(xv_v, [iv]) * _SCALE
        o01_v[0, pl.ds(i * 16, 16)] = plsc.load_gather(xi_v, [zeros, iv])
        o01_v[1, pl.ds(i * 16, 16)] = plsc.load_gather(xi_v, [ones, iv])

    @pl.when(wid != _NW - 1)
    def _full_write():
        cp_rc = pltpu.async_copy(o01_v, rc_hbm.at[:, pl.ds(obase, _OPT)], sem_out)
        cp_v = pltpu.async_copy(ov_v, val_hbm.at[pl.ds(obase, _OPT)], sem_out)
        cp_rc.wait()
        cp_v.wait()

    @pl.when(wid == _NW - 1)
    def _tail_write():
        cp_rc = pltpu.async_copy(o01_v.at[:, pl.ds(0, _CTAIL)],
                                 rc_hbm.at[:, pl.ds(obase, _CTAIL)], sem_out)
        cp_v = pltpu.async_copy(ov_v.at[pl.ds(0, _TAIL)],
                                val_hbm.at[pl.ds(obase, _TAIL)], sem_out)
        cp_rc.wait()
        cp_v.wait()


def kernel(x_indices, x_values):
    xi = x_indices.astype(jnp.int32)
    lidx = jnp.asarray(_LIDX)
    rc_pad, val = _compact(xi, x_values, lidx)
    return rc_pad[:, :_K].astype(x_indices.dtype), val


# speedup vs baseline: 1.1742x; 1.0033x over previous
import functools

import numpy as np
import jax
import jax.numpy as jnp
from jax import lax
from jax.experimental import pallas as pl
from jax.experimental.pallas import tpu as pltpu
from jax.experimental.pallas import tpu_sc as plsc

_N = 16384
_NNZ = 262144
_DPROB = 0.5
_SCALE = np.float32(1.0 / (1.0001 - _DPROB))


def _tf2x32_np(k0, k1, x0, x1):
    rot1 = (13, 15, 26, 6)
    rot2 = (17, 29, 16, 24)
    k0 = np.uint32(k0)
    k1 = np.uint32(k1)
    ks = (k0, k1, np.uint32(k0 ^ k1 ^ np.uint32(0x1BD11BDA)))
    x0 = x0.astype(np.uint32) + k0
    x1 = x1.astype(np.uint32) + k1

    def rnds(x0, x1, rots):
        for r in rots:
            x0 = x0 + x1
            x1 = (x1 << np.uint32(r)) | (x1 >> np.uint32(32 - r))
            x1 = x0 ^ x1
        return x0, x1

    for i in range(5):
        x0, x1 = rnds(x0, x1, rot1 if i % 2 == 0 else rot2)
        x0 = x0 + ks[(i + 1) % 3]
        x1 = x1 + ks[(i + 2) % 3] + np.uint32(i + 1)
    return x0, x1


def _fixed_mask_np(nnz, dprob):
    o0, o1 = _tf2x32_np(0, 0, np.array([0], np.uint32), np.array([123], np.uint32))
    b0, b1 = _tf2x32_np(o0[0], o1[0], np.zeros(nnz, np.uint32),
                        np.arange(nnz, dtype=np.uint32))
    bits = b0 ^ b1
    float_bits = (bits >> np.uint32(9)) | np.uint32(0x3F800000)
    u = np.maximum(np.float32(0.0), float_bits.view(np.float32) - np.float32(1.0))
    return np.floor(u + np.float32(1.0 - dprob)).astype(bool)


_msk = _fixed_mask_np(_NNZ, _DPROB)
_sel = np.nonzero(_msk)[0].astype(np.int64)
_K = int(_sel.shape[0])

_NW = 32
_OPT = 4224
_PAD = _NW * _OPT
_STRIDE = 8448
_OFF = -1024
_IN_LEN = 9344
_HL = 4608
_J1 = 112

_TAIL = _K - (_NW - 1) * _OPT
_PADC = ((_K + 127) // 128) * 128
_CTAIL = _PADC - (_NW - 1) * _OPT
assert 0 < _TAIL <= _OPT and (_NW - 1) * _OPT % 128 == 0 and _CTAIL % 128 == 0

_selp = np.concatenate([_sel, np.full(_PAD - _K, _sel[-1], np.int64)])
_in_lo_np = np.clip(_OFF + np.arange(_NW) * _STRIDE, 0, _NNZ - _IN_LEN)
_lidx_np = (_selp.reshape(_NW, _OPT) - _in_lo_np[:, None]).astype(np.int32)
assert _lidx_np.min() >= 0 and _lidx_np.max() < _IN_LEN
_LIDX = _lidx_np.reshape(_PAD)


@functools.partial(
    pl.kernel,
    mesh=plsc.VectorSubcoreMesh(core_axis_name="c", subcore_axis_name="s"),
    compiler_params=pltpu.CompilerParams(needs_layout_passes=False),
    out_type=[
        jax.ShapeDtypeStruct((2, _PADC), jnp.int32),
        jax.ShapeDtypeStruct((_K,), jnp.float32),
    ],
    scratch_types=[
        pltpu.VMEM((2, _IN_LEN), jnp.int32),
        pltpu.VMEM((_IN_LEN,), jnp.float32),
        pltpu.VMEM((_OPT,), jnp.int32),
        pltpu.VMEM((2, _OPT), jnp.int32),
        pltpu.VMEM((_OPT,), jnp.float32),
        pltpu.SemaphoreType.DMA,
        pltpu.SemaphoreType.DMA,
        pltpu.SemaphoreType.DMA,
    ],
)
def _compact(xi_hbm, xv_hbm, lidx_hbm, rc_hbm, val_hbm,
             xi_v, xv_v, lidx_v, o01_v, ov_v, sem_in, sem_in2, sem_out):
    c = lax.axis_index("c")
    s = lax.axis_index("s")
    wid = s * 2 + c
    in_lo = pl.multiple_of(jnp.clip(_OFF + wid * _STRIDE, 0, _NNZ - _IN_LEN), 128)
    obase = pl.multiple_of(wid * _OPT, 128)

    cp_xi = pltpu.async_copy(xi_hbm.at[:, pl.ds(in_lo, _IN_LEN)], xi_v, sem_in)
    cp_xv = pltpu.async_copy(xv_hbm.at[pl.ds(in_lo, _IN_LEN)], xv_v, sem_in)
    cp_li = pltpu.async_copy(lidx_hbm.at[pl.ds(obase, _OPT)], lidx_v, sem_in)
    cp_xi.wait()
    cp_xv.wait()
    cp_li.wait()

    zeros = jnp.zeros((16,), jnp.int32)
    ones = jnp.ones((16,), jnp.int32)

    @plsc.parallel_loop(0, _OPT // 16, unroll=8)
    def _gather(i):
        iv = lidx_v[pl.ds(i * 16, 16)]
        ov_v[pl.ds(i * 16, 16)] = plsc.load_gather(xv_v, [iv]) * _SCALE
        o01_v[0, pl.ds(i * 16, 16)] = plsc.load_gather(xi_v, [zeros, iv])
        o01_v[1, pl.ds(i * 16, 16)] = plsc.load_gather(xi_v, [ones, iv])

    @pl.when(wid != _NW - 1)
    def _full_write():
        cp_rc = pltpu.async_copy(o01_v, rc_hbm.at[:, pl.ds(obase, _OPT)], sem_out)
        cp_v = pltpu.async_copy(ov_v, val_hbm.at[pl.ds(obase, _OPT)], sem_out)
        cp_rc.wait()
        cp_v.wait()

    @pl.when(wid == _NW - 1)
    def _tail_write():
        cp_rc = pltpu.async_copy(o01_v.at[:, pl.ds(0, _CTAIL)],
                                 rc_hbm.at[:, pl.ds(obase, _CTAIL)], sem_out)
        cp_v = pltpu.async_copy(ov_v.at[pl.ds(0, _TAIL)],
                                val_hbm.at[pl.ds(obase, _TAIL)], sem_out)
        cp_rc.wait()
        cp_v.wait()


_LIDX_DEV = None


def kernel(x_indices, x_values):
    global _LIDX_DEV
    if _LIDX_DEV is None:
        _LIDX_DEV = jnp.asarray(_LIDX)
    xi = x_indices.astype(jnp.int32)
    rc_pad, val = _compact(xi, x_values, _LIDX_DEV)
    return rc_pad[:, :_K].astype(x_indices.dtype), val
